# baseline (speedup 1.0000x reference)
import jax
import jax.numpy as jnp
from jax import lax
from jax.experimental import pallas as pl
from jax.experimental.pallas import tpu as pltpu
from jax.experimental.pallas import tpu_sc as plsc

B = 4096
L = 200
DIM = 64
N = B * L

NC = 2
NS = 16
NW = NC * NS
B_PER_W = N // NW
CHUNK = 1024
STEPS = B_PER_W // CHUNK


def _sc_gather(table, idx):
    mesh = plsc.VectorSubcoreMesh(core_axis_name="c", subcore_axis_name="s")

    @pl.kernel(
        out_type=jax.ShapeDtypeStruct((N, DIM), jnp.float32),
        mesh=mesh,
        scratch_types=[
            pltpu.VMEM((CHUNK,), jnp.int32),
            pltpu.VMEM((CHUNK, DIM), jnp.float32),
            pltpu.SemaphoreType.DMA,
        ],
        compiler_params=pltpu.CompilerParams(use_tc_tiling_on_sc=False),
    )
    def gather_kernel(table_hbm, idx_hbm, out_hbm, idx_v, rows_v, sem):
        wid = lax.axis_index("s") * NC + lax.axis_index("c")
        base = wid * B_PER_W

        @pl.loop(0, STEPS)
        def _(step):
            off = base + step * CHUNK
            pltpu.sync_copy(idx_hbm.at[pl.ds(off, CHUNK)], idx_v)
            pltpu.async_copy(table_hbm.at[idx_v], rows_v, sem).wait()
            pltpu.sync_copy(rows_v, out_hbm.at[pl.ds(off, CHUNK)])

    return gather_kernel(table, idx)


def kernel(x, table):
    idx = x.reshape(N).astype(jnp.int32)
    out = _sc_gather(table, idx)
    return out.reshape(B, L, DIM)


# speedup vs baseline: 1.0070x; 1.0070x over previous
import jax
import jax.numpy as jnp
from jax import lax
from jax.experimental import pallas as pl
from jax.experimental.pallas import tpu as pltpu
from jax.experimental.pallas import tpu_sc as plsc

B = 4096
L = 200
DIM = 64
N = B * L

NC = 2
NS = 16
NW = NC * NS
B_PER_W = N // NW
NBUF = 4
CHUNK = 400
STEPS = B_PER_W // CHUNK


def _sc_gather(table, idx):
    mesh = plsc.VectorSubcoreMesh(core_axis_name="c", subcore_axis_name="s")

    @pl.kernel(
        out_type=jax.ShapeDtypeStruct((N, DIM), jnp.float32),
        mesh=mesh,
        scratch_types=[
            pltpu.VMEM((NBUF, CHUNK), jnp.int32),
            pltpu.VMEM((NBUF, CHUNK, DIM), jnp.float32),
            pltpu.SemaphoreType.DMA((NBUF,)),
        ],
        compiler_params=pltpu.CompilerParams(use_tc_tiling_on_sc=False),
    )
    def gather_kernel(table_hbm, idx_hbm, out_hbm, idx_v, rows_v, sems):
        wid = lax.axis_index("s") * NC + lax.axis_index("c")
        base = wid * B_PER_W

        def fire(b, chunk_i):
            off = base + chunk_i * CHUNK
            pltpu.sync_copy(idx_hbm.at[pl.ds(off, CHUNK)], idx_v.at[b])
            pltpu.async_copy(table_hbm.at[idx_v.at[b]], rows_v.at[b],
                             sems.at[b])

        def drain_store(b, chunk_i):
            pltpu.make_async_copy(table_hbm.at[idx_v.at[b]], rows_v.at[b],
                                  sems.at[b]).wait()
            off = base + chunk_i * CHUNK
            pltpu.sync_copy(rows_v.at[b], out_hbm.at[pl.ds(off, CHUNK)])

        for b in range(NBUF):
            fire(b, b)

        @pl.loop(0, STEPS // NBUF - 1)
        def _(h):
            for b in range(NBUF):
                i = h * NBUF + b
                drain_store(b, i)
                fire(b, i + NBUF)

        for b in range(NBUF):
            drain_store(b, STEPS - NBUF + b)

    return gather_kernel(table, idx)


def kernel(x, table):
    idx = x.reshape(N).astype(jnp.int32)
    out = _sc_gather(table, idx)
    return out.reshape(B, L, DIM)
